# baseline (speedup 1.0000x reference)
import functools

import jax
import jax.numpy as jnp
from jax import lax
from jax.experimental import pallas as pl
from jax.experimental.pallas import tpu as pltpu
from jax.experimental.pallas import tpu_sc as plsc

_D = 768
_F = 2048
_E = 8
_R = 8
_N = 2048
_FT = 512


def _logits_kernel(x_ref, gw_ref, logits_ref):
    logits_ref[...] = jax.lax.dot_general(
        x_ref[...], gw_ref[...], (((1,), (1,)), ((), ())),
        preferred_element_type=jnp.float32)


def _run_logits(data, gate_w):
    return pl.pallas_call(
        _logits_kernel,
        out_shape=jax.ShapeDtypeStruct((_N, _E), jnp.float32),
    )(data, gate_w)


_INFO = plsc.get_sparse_core_info()
_NW = _INFO.num_cores * _INFO.num_subcores
_TPW = _N // _NW
_LANES = 16
_NCH = _TPW // _LANES


def _topk_sc_kernel(lt_hbm, wt_hbm, lv, ov):
    wid = lax.axis_index("s") * _INFO.num_cores + lax.axis_index("c")
    pltpu.sync_copy(lt_hbm.at[wid], lv)
    for c in range(_NCH):
        m1 = lv[pl.ds(c * _LANES, _LANES)]
        i1 = jnp.zeros((_LANES,), jnp.int32)
        for e in range(1, _E):
            v = lv[pl.ds(e * _TPW + c * _LANES, _LANES)]
            gt = v > m1
            i1 = jnp.where(gt, e, i1)
            m1 = jnp.maximum(m1, v)
        m2 = jnp.full((_LANES,), -jnp.inf, jnp.float32)
        i2 = jnp.zeros((_LANES,), jnp.int32)
        for e in range(_E):
            v = lv[pl.ds(e * _TPW + c * _LANES, _LANES)]
            v = jnp.where(i1 == e, -jnp.inf, v)
            gt = v > m2
            i2 = jnp.where(gt, e, i2)
            m2 = jnp.maximum(m2, v)
        v2 = jnp.exp(m2 - m1)
        inv = 1.0 / (1.0 + v2)
        w1 = inv
        w2 = v2 * inv
        zero = jnp.zeros((_LANES,), jnp.float32)
        for e in range(_E):
            ov[pl.ds(e * _TPW + c * _LANES, _LANES)] = (
                jnp.where(i1 == e, w1, zero) + jnp.where(i2 == e, w2, zero))
    pltpu.sync_copy(ov, wt_hbm.at[wid])


def _run_topk_sc(lt_grouped):
    mesh = plsc.VectorSubcoreMesh(core_axis_name="c", subcore_axis_name="s")
    k = functools.partial(
        pl.kernel,
        mesh=mesh,
        out_type=jax.ShapeDtypeStruct((_NW, _E * _TPW), jnp.float32),
        scratch_types=[
            pltpu.VMEM((_E * _TPW,), jnp.float32),
            pltpu.VMEM((_E * _TPW,), jnp.float32),
        ],
    )(_topk_sc_kernel)
    return k(lt_grouped)


def _mlp_kernel(x_ref, wd_ref, w1_ref, w3_ref, w2_ref, af_ref, bf_ref,
                ex_ref, out_ref, xb_ref):
    j = pl.program_id(0)

    @pl.when(j == 0)
    def _prologue():
        x = x_ref[...]
        xb_ref[...] = x.astype(jnp.bfloat16)
        wexp = jax.lax.dot_general(
            wd_ref[...], ex_ref[...], (((1,), (0,)), ((), ())),
            preferred_element_type=jnp.float32)
        t = jax.lax.dot_general(
            x, af_ref[...], (((1,), (1,)), ((), ())),
            preferred_element_type=jnp.float32)
        out_ref[...] = jax.lax.dot_general(
            wexp * t, bf_ref[...], (((1,), (0,)), ((), ())),
            preferred_element_type=jnp.float32)

    xb = xb_ref[...]
    h1 = jax.lax.dot_general(
        xb, w1_ref[...].astype(jnp.bfloat16), (((1,), (1,)), ((), ())),
        preferred_element_type=jnp.float32)
    h3 = jax.lax.dot_general(
        xb, w3_ref[...].astype(jnp.bfloat16), (((1,), (1,)), ((), ())),
        preferred_element_type=jnp.float32)
    h = (h1 * jax.nn.sigmoid(h1)) * h3
    out_ref[...] += jax.lax.dot_general(
        h.astype(jnp.bfloat16), w2_ref[...].astype(jnp.bfloat16),
        (((1,), (1,)), ((), ())),
        preferred_element_type=jnp.float32)


def kernel(data, gate_w, w1, w3, w2, lora_A, lora_B):
    a_flat = lora_A.reshape(_E * _R, _D)
    b_flat = lora_B.transpose(0, 2, 1).reshape(_E * _R, _D)
    expand = jnp.repeat(jnp.eye(_E, dtype=jnp.float32), _R, axis=1)

    logits = _run_logits(data, gate_w)
    lt_g = (logits.reshape(_NW, _TPW, _E).transpose(0, 2, 1)
            .reshape(_NW, _E * _TPW))
    wd_g = _run_topk_sc(lt_g)
    wdense = (wd_g.reshape(_NW, _E, _TPW).transpose(0, 2, 1)
              .reshape(_N, _E))

    grid = (_F // _FT,)
    out = pl.pallas_call(
        _mlp_kernel,
        grid=grid,
        in_specs=[
            pl.BlockSpec((_N, _D), lambda j: (0, 0)),
            pl.BlockSpec((_N, _E), lambda j: (0, 0)),
            pl.BlockSpec((_FT, _D), lambda j: (j, 0)),
            pl.BlockSpec((_FT, _D), lambda j: (j, 0)),
            pl.BlockSpec((_D, _FT), lambda j: (0, j)),
            pl.BlockSpec((_E * _R, _D), lambda j: (0, 0)),
            pl.BlockSpec((_E * _R, _D), lambda j: (0, 0)),
            pl.BlockSpec((_E, _E * _R), lambda j: (0, 0)),
        ],
        out_specs=pl.BlockSpec((_N, _D), lambda j: (0, 0)),
        out_shape=jax.ShapeDtypeStruct((_N, _D), jnp.float32),
        scratch_shapes=[pltpu.VMEM((_N, _D), jnp.bfloat16)],
    )(data, wdense, w1, w3, w2, a_flat, b_flat, expand)
    return out, logits


# speedup vs baseline: 1.5713x; 1.5713x over previous
import jax
import jax.numpy as jnp
from jax.experimental import pallas as pl
from jax.experimental.pallas import tpu as pltpu

_D = 768
_F = 2048
_E = 8
_R = 8
_N = 2048
_FT = 512


def _fused_kernel(x_ref, gw_ref, w1_ref, w3_ref, w2_ref, af_ref, bf_ref,
                  ex_ref, out_ref, logits_ref, xb_ref):
    j = pl.program_id(0)

    @pl.when(j == 0)
    def _prologue():
        x = x_ref[...]
        xb_ref[...] = x.astype(jnp.bfloat16)

        logits = jax.lax.dot_general(
            x, gw_ref[...], (((1,), (1,)), ((), ())),
            preferred_element_type=jnp.float32)
        logits_ref[...] = logits

        eidx = jax.lax.broadcasted_iota(jnp.int32, logits.shape, 1)
        m1 = jnp.max(logits, axis=-1, keepdims=True)
        i1 = jnp.min(jnp.where(logits == m1, eidx, _E), axis=-1,
                     keepdims=True)
        sel1 = eidx == i1
        masked = jnp.where(sel1, -jnp.inf, logits)
        m2 = jnp.max(masked, axis=-1, keepdims=True)
        i2 = jnp.min(jnp.where(masked == m2, eidx, _E), axis=-1,
                     keepdims=True)
        sel2 = eidx == i2
        v2 = jnp.exp(m2 - m1)
        inv = 1.0 / (1.0 + v2)
        wdense = jnp.where(sel1, inv, 0.0) + jnp.where(sel2, v2 * inv, 0.0)

        wexp = jax.lax.dot_general(
            wdense, ex_ref[...], (((1,), (0,)), ((), ())),
            preferred_element_type=jnp.float32)

        t = jax.lax.dot_general(
            x, af_ref[...], (((1,), (1,)), ((), ())),
            preferred_element_type=jnp.float32)
        out_ref[...] = jax.lax.dot_general(
            wexp * t, bf_ref[...], (((1,), (0,)), ((), ())),
            preferred_element_type=jnp.float32)

    xb = xb_ref[...]
    h1 = jax.lax.dot_general(
        xb, w1_ref[...].astype(jnp.bfloat16), (((1,), (1,)), ((), ())),
        preferred_element_type=jnp.float32)
    h3 = jax.lax.dot_general(
        xb, w3_ref[...].astype(jnp.bfloat16), (((1,), (1,)), ((), ())),
        preferred_element_type=jnp.float32)
    h = (h1 * jax.nn.sigmoid(h1)) * h3
    out_ref[...] += jax.lax.dot_general(
        h.astype(jnp.bfloat16), w2_ref[...].astype(jnp.bfloat16),
        (((1,), (1,)), ((), ())),
        preferred_element_type=jnp.float32)


def kernel(data, gate_w, w1, w3, w2, lora_A, lora_B):
    a_flat = lora_A.reshape(_E * _R, _D)
    b_flat = lora_B.transpose(0, 2, 1).reshape(_E * _R, _D)
    expand = jnp.repeat(jnp.eye(_E, dtype=jnp.float32), _R, axis=1)

    grid = (_F // _FT,)
    out, logits = pl.pallas_call(
        _fused_kernel,
        grid=grid,
        in_specs=[
            pl.BlockSpec((_N, _D), lambda j: (0, 0)),
            pl.BlockSpec((_E, _D), lambda j: (0, 0)),
            pl.BlockSpec((_FT, _D), lambda j: (j, 0)),
            pl.BlockSpec((_FT, _D), lambda j: (j, 0)),
            pl.BlockSpec((_D, _FT), lambda j: (0, j)),
            pl.BlockSpec((_E * _R, _D), lambda j: (0, 0)),
            pl.BlockSpec((_E * _R, _D), lambda j: (0, 0)),
            pl.BlockSpec((_E, _E * _R), lambda j: (0, 0)),
        ],
        out_specs=[
            pl.BlockSpec((_N, _D), lambda j: (0, 0)),
            pl.BlockSpec((_N, _E), lambda j: (0, 0)),
        ],
        out_shape=[
            jax.ShapeDtypeStruct((_N, _D), jnp.float32),
            jax.ShapeDtypeStruct((_N, _E), jnp.float32),
        ],
        scratch_shapes=[pltpu.VMEM((_N, _D), jnp.bfloat16)],
    )(data, gate_w, w1, w3, w2, a_flat, b_flat, expand)
    return out, logits


# speedup vs baseline: 1.5983x; 1.0172x over previous
import jax
import jax.numpy as jnp
from jax.experimental import pallas as pl
from jax.experimental.pallas import tpu as pltpu

_D = 768
_F = 2048
_E = 8
_R = 8
_N = 2048
_FT = 512


def _fused_kernel(x_ref, gw_ref, w1_ref, w3_ref, w2_ref, af_ref, bf_ref,
                  ex_ref, out_ref, logits_ref, xb_ref):
    j = pl.program_id(0)

    @pl.when(j == 0)
    def _prologue():
        x = x_ref[...]
        xb_ref[...] = x.astype(jnp.bfloat16)

        logits = jax.lax.dot_general(
            x, gw_ref[...], (((1,), (1,)), ((), ())),
            preferred_element_type=jnp.float32)
        logits_ref[...] = logits

        eidx = jax.lax.broadcasted_iota(jnp.int32, logits.shape, 1)
        m1 = jnp.max(logits, axis=-1, keepdims=True)
        i1 = jnp.min(jnp.where(logits == m1, eidx, _E), axis=-1,
                     keepdims=True)
        sel1 = eidx == i1
        masked = jnp.where(sel1, -jnp.inf, logits)
        m2 = jnp.max(masked, axis=-1, keepdims=True)
        i2 = jnp.min(jnp.where(masked == m2, eidx, _E), axis=-1,
                     keepdims=True)
        sel2 = eidx == i2
        v2 = jnp.exp(m2 - m1)
        inv = 1.0 / (1.0 + v2)
        wdense = jnp.where(sel1, inv, 0.0) + jnp.where(sel2, v2 * inv, 0.0)

        wexp = jax.lax.dot_general(
            wdense, ex_ref[...], (((1,), (0,)), ((), ())),
            preferred_element_type=jnp.float32)

        t = jax.lax.dot_general(
            x, af_ref[...], (((1,), (1,)), ((), ())),
            preferred_element_type=jnp.float32)
        out_ref[...] = jax.lax.dot_general(
            wexp * t, bf_ref[...], (((1,), (0,)), ((), ())),
            preferred_element_type=jnp.float32)

    @pl.when(j > 0)
    def _mlp_step():
        xb = xb_ref[...]
        h1 = jax.lax.dot_general(
            xb, w1_ref[...].astype(jnp.bfloat16), (((1,), (1,)), ((), ())),
            preferred_element_type=jnp.float32)
        h3 = jax.lax.dot_general(
            xb, w3_ref[...].astype(jnp.bfloat16), (((1,), (1,)), ((), ())),
            preferred_element_type=jnp.float32)
        h = (h1 * jax.nn.sigmoid(h1)) * h3
        out_ref[...] += jax.lax.dot_general(
            h.astype(jnp.bfloat16), w2_ref[...].astype(jnp.bfloat16),
            (((1,), (1,)), ((), ())),
            preferred_element_type=jnp.float32)


def kernel(data, gate_w, w1, w3, w2, lora_A, lora_B):
    a_flat = lora_A.reshape(_E * _R, _D)
    b_flat = lora_B.transpose(0, 2, 1).reshape(_E * _R, _D)
    expand = jnp.repeat(jnp.eye(_E, dtype=jnp.float32), _R, axis=1)

    grid = (_F // _FT + 1,)
    out, logits = pl.pallas_call(
        _fused_kernel,
        grid=grid,
        in_specs=[
            pl.BlockSpec((_N, _D), lambda j: (0, 0)),
            pl.BlockSpec((_E, _D), lambda j: (0, 0)),
            pl.BlockSpec((_FT, _D),
                         lambda j: (jnp.maximum(j - 1, 0), 0)),
            pl.BlockSpec((_FT, _D),
                         lambda j: (jnp.maximum(j - 1, 0), 0)),
            pl.BlockSpec((_D, _FT),
                         lambda j: (0, jnp.maximum(j - 1, 0))),
            pl.BlockSpec((_E * _R, _D), lambda j: (0, 0)),
            pl.BlockSpec((_E * _R, _D), lambda j: (0, 0)),
            pl.BlockSpec((_E, _E * _R), lambda j: (0, 0)),
        ],
        out_specs=[
            pl.BlockSpec((_N, _D), lambda j: (0, 0)),
            pl.BlockSpec((_N, _E), lambda j: (0, 0)),
        ],
        out_shape=[
            jax.ShapeDtypeStruct((_N, _D), jnp.float32),
            jax.ShapeDtypeStruct((_N, _E), jnp.float32),
        ],
        scratch_shapes=[pltpu.VMEM((_N, _D), jnp.bfloat16)],
    )(data, gate_w, w1, w3, w2, a_flat, b_flat, expand)
    return out, logits
